# Initial kernel scaffold; baseline (speedup 1.0000x reference)
import jax
import jax.numpy as jnp
from jax.experimental import pallas as pl


def kernel(x, edge_index, W_in, b_in, W_blocks, b_blocks, W_out, b_out):
    raise NotImplementedError("write your pallas kernel here")


# speedup vs baseline: 1.0153x; 1.0153x over previous
import functools

import jax
import jax.numpy as jnp
from jax import lax
from jax.experimental import pallas as pl

N_NODES = 10000
N_EDGES = 320000
HID = 128
LEAK = 0.2


def _leaky(v):
    return jnp.maximum(v, LEAK * v)


def _mm_body(x_ref, w_ref, b_ref, o_ref, *, act):
    acc = jnp.dot(x_ref[...], w_ref[...], preferred_element_type=jnp.float32)
    acc = acc + b_ref[...]
    o_ref[...] = _leaky(acc) if act else acc


def _matmul(x, w, b, act, m_block=2000):
    m, k = x.shape
    n = w.shape[1]
    grid = (m // m_block,)
    return pl.pallas_call(
        functools.partial(_mm_body, act=act),
        grid=grid,
        in_specs=[
            pl.BlockSpec((m_block, k), lambda i: (i, 0)),
            pl.BlockSpec((k, n), lambda i: (0, 0)),
            pl.BlockSpec((1, n), lambda i: (0, 0)),
        ],
        out_specs=pl.BlockSpec((m_block, n), lambda i: (i, 0)),
        out_shape=jax.ShapeDtypeStruct((m, n), jnp.float32),
    )(x, w, b.reshape(1, n))


def kernel(x, edge_index, W_in, b_in, W_blocks, b_blocks, W_out, b_out):
    src = edge_index[0].astype(jnp.int32)
    dst = edge_index[1].astype(jnp.int32)

    h = _matmul(x, W_in, b_in, act=True)

    n_block = W_blocks.shape[0]
    for i in range(n_block):
        Wt = W_blocks[i, :HID, :]
        Wb = W_blocks[i, HID:, :]
        Wcat = jnp.concatenate([Wb, Wt - Wb], axis=1)
        bcat = jnp.concatenate([jnp.zeros_like(b_blocks[i]), b_blocks[i]])
        AB = _matmul(h, Wcat, bcat, act=False)
        A, B = AB[:, :HID], AB[:, HID:]
        msg = _leaky(jnp.take(A, src, axis=0) + jnp.take(B, dst, axis=0))
        agg = jax.ops.segment_max(msg, dst, num_segments=N_NODES)
        agg = jnp.where(jnp.isneginf(agg), 0.0, agg)
        h = h + agg

    out = _matmul(h, W_out, b_out, act=False)
    return out
